# baseline (speedup 1.0000x reference)
import jax
import jax.numpy as jnp
from jax.experimental import pallas as pl
from jax.experimental.pallas import tpu as pltpu

B, T, N, H = 4, 16, 10000, 64
TB = 4
TT = T // TB


def _body(sp_ref, *refs):
    stack_refs = refs[:TB]
    hid_ref, hint_ref, spv_ref, out_ref, ptr_ref = refs[TB:]
    b = pl.program_id(0)
    tb = pl.program_id(1)
    tgt = sp_ref[b] + 1
    base = TB * tb

    for j in range(TB):
        @pl.when(tgt == base + j)
        def _():
            out_ref[:, j:j + 1] = hid_ref[...]

        @pl.when(tgt != base + j)
        def _():
            out_ref[:, j:j + 1] = stack_refs[j][...]

    @pl.when((b == 0) & (tb == 0))
    def _():
        h = hint_ref[...]
        a0 = h[:, :, 0]
        a1 = h[:, :, 1]
        a2 = h[:, :, 2]
        ops = jnp.where(a0 >= a1,
                        jnp.where(a0 >= a2, 0, 2),
                        jnp.where(a1 >= a2, 1, 2)).astype(jnp.int32)
        ptr_ref[...] = jnp.maximum(spv_ref[...] + ops - 1, 0)


def kernel(stack, stack_pointers, hint_preds, hiddens, graph_fts):
    del graph_fts
    sp_flat = jnp.reshape(stack_pointers, (B,))
    stack_t = jnp.transpose(stack, (0, 1, 3, 2))
    hid_t = jnp.transpose(hiddens, (0, 1, 3, 2))

    def mk_stack_idx(j):
        def stack_idx(b, tb, sp):
            t = TB * tb + j
            tt = jnp.where(t == sp[b] + 1, jnp.maximum(t - TB, 0), t)
            return (b, tt, 0, 0)
        return stack_idx

    grid_spec = pltpu.PrefetchScalarGridSpec(
        num_scalar_prefetch=1,
        grid=(B, TT),
        in_specs=(
            [pl.BlockSpec((1, 1, H, N), mk_stack_idx(j)) for j in range(TB)]
            + [
                pl.BlockSpec((1, 1, H, N), lambda b, tb, sp: (0, b, 0, 0)),
                pl.BlockSpec((1, B, 3), lambda b, tb, sp: (0, 0, 0)),
                pl.BlockSpec((1, B), lambda b, tb, sp: (0, 0)),
            ]
        ),
        out_specs=[
            pl.BlockSpec((1, TB, H, N), lambda b, tb, sp: (b, tb, 0, 0)),
            pl.BlockSpec((1, B), lambda b, tb, sp: (0, 0)),
        ],
    )

    new_stack_t, new_ptrs = pl.pallas_call(
        _body,
        grid_spec=grid_spec,
        out_shape=[
            jax.ShapeDtypeStruct((B, T, H, N), jnp.float32),
            jax.ShapeDtypeStruct((1, B), jnp.int32),
        ],
    )(sp_flat, *([stack_t] * TB), hid_t, hint_preds, stack_pointers)
    return (jnp.transpose(new_stack_t, (0, 1, 3, 2)), new_ptrs)
